# baseline (speedup 1.0000x reference)
import jax
import jax.numpy as jnp
from jax.experimental import pallas as pl

EMB = 2048
NEXP = 16
ROWS_PER_BLOCK = 1024
PACK = 32


def _gating_kernel(x_ref, wt_ref, out_ref):
    x = x_ref[0]
    wt = wt_ref[...]
    logits = jnp.dot(x, wt, preferred_element_type=jnp.float32)
    lt = logits.T

    m = jnp.max(lt, axis=0, keepdims=True)
    e = jnp.exp(lt - m)
    s = jnp.sum(e, axis=0, keepdims=True)
    p = e / s

    iota = jax.lax.broadcasted_iota(jnp.int32, p.shape, 0)
    w1 = jnp.max(p, axis=0, keepdims=True)
    i1 = jnp.min(jnp.where(p == w1, iota, NEXP), axis=0, keepdims=True)
    masked = jnp.where(iota == i1, -1.0, p)
    w2 = jnp.max(masked, axis=0, keepdims=True)
    i2 = jnp.min(jnp.where(masked == w2, iota, NEXP), axis=0, keepdims=True)

    t = jnp.exp(w2 - w1)
    denom = 1.0 + t
    packed = jnp.concatenate(
        [
            p,
            1.0 / denom,
            t / denom,
            i1.astype(jnp.float32),
            i2.astype(jnp.float32),
            jnp.zeros((PACK - NEXP - 4, p.shape[1]), jnp.float32),
        ],
        axis=0,
    )
    out_ref[0] = packed.T


def kernel(x, W):
    B, S, D = x.shape
    wt = W.T
    R = ROWS_PER_BLOCK
    SB = S // R
    grid = (B * SB,)

    out = pl.pallas_call(
        _gating_kernel,
        grid=grid,
        in_specs=[
            pl.BlockSpec((1, R, D), lambda i: (i // SB, i % SB, 0)),
            pl.BlockSpec((D, NEXP), lambda i: (0, 0)),
        ],
        out_specs=pl.BlockSpec((1, R, PACK), lambda i: (i // SB, i % SB, 0)),
        out_shape=jax.ShapeDtypeStruct((B, S, PACK), jnp.float32),
    )(x, wt)

    gw = out[..., :NEXP]
    tkw = out[..., NEXP:NEXP + 2]
    tki = out[..., NEXP + 2:NEXP + 4].astype(jnp.int32)
    return (gw, tkw, tki)


# speedup vs baseline: 1.0740x; 1.0740x over previous
import jax
import jax.numpy as jnp
from jax.experimental import pallas as pl

EMB = 2048
NEXP = 16
ROWS_PER_BLOCK = 1024


def _gating_kernel(x_ref, wt_ref, gw_ref, tkw_ref, tki_ref):
    x = x_ref[0]
    wt = wt_ref[...]
    logits = jnp.dot(x, wt, preferred_element_type=jnp.float32)
    lt = logits.T

    m = jnp.max(lt, axis=0, keepdims=True)
    e = jnp.exp(lt - m)
    s = jnp.sum(e, axis=0, keepdims=True)
    p = e / s
    gw_ref[0] = p.T

    iota = jax.lax.broadcasted_iota(jnp.int32, p.shape, 0)
    w1 = jnp.max(p, axis=0, keepdims=True)
    i1 = jnp.min(jnp.where(p == w1, iota, NEXP), axis=0, keepdims=True)
    masked = jnp.where(iota == i1, -1.0, p)
    w2 = jnp.max(masked, axis=0, keepdims=True)
    i2 = jnp.min(jnp.where(masked == w2, iota, NEXP), axis=0, keepdims=True)

    t = jnp.exp(w2 - w1)
    denom = 1.0 + t
    tkw_ref[0] = jnp.concatenate([1.0 / denom, t / denom], axis=0).T
    tki_ref[0] = jnp.concatenate([i1, i2], axis=0).T.astype(jnp.int32)


def kernel(x, W):
    B, S, D = x.shape
    wt = W.T
    R = ROWS_PER_BLOCK
    SB = S // R
    grid = (B * SB,)

    gw, tkw, tki = pl.pallas_call(
        _gating_kernel,
        grid=grid,
        in_specs=[
            pl.BlockSpec((1, R, D), lambda i: (i // SB, i % SB, 0)),
            pl.BlockSpec((D, NEXP), lambda i: (0, 0)),
        ],
        out_specs=[
            pl.BlockSpec((1, R, NEXP), lambda i: (i // SB, i % SB, 0)),
            pl.BlockSpec((1, R, 2), lambda i: (i // SB, i % SB, 0)),
            pl.BlockSpec((1, R, 2), lambda i: (i // SB, i % SB, 0)),
        ],
        out_shape=[
            jax.ShapeDtypeStruct((B, S, NEXP), jnp.float32),
            jax.ShapeDtypeStruct((B, S, 2), jnp.float32),
            jax.ShapeDtypeStruct((B, S, 2), jnp.int32),
        ],
    )(x, wt)

    return (gw, tkw, tki)
